# baseline (speedup 1.0000x reference)
import functools

import jax
import jax.numpy as jnp
from jax import lax
from jax.experimental import pallas as pl
from jax.experimental.pallas import tpu as pltpu
from jax.experimental.pallas import tpu_sc as plsc

C = 128
NHEAD = 8
HD = C // NHEAD
S = 14
NHID = 128


def _dot_t(x, w):
    return lax.dot_general(x, w, (((1,), (1,)), ((), ())),
                           preferred_element_type=jnp.float32)


def _ln(x, g, b, eps=1e-5):
    m = jnp.mean(x, -1, keepdims=True)
    v = jnp.mean((x - m) * (x - m), -1, keepdims=True)
    return (x - m) / jnp.sqrt(v + eps) * g + b


def _tab_body(x_ref, wq, wk, wv, wo, w1, w2, bq, bk, bv, bo, b1, b2,
              ln1g, ln1b, ln2g, ln2b, tng, tnb, o_ref):
    x = x_ref[...]
    bb = x.shape[0]
    xf = x.reshape(bb * S, C)
    q = (_dot_t(xf, wq[...]) + bq[...]) * (1.0 / jnp.sqrt(jnp.float32(HD)))
    k = _dot_t(xf, wk[...]) + bk[...]
    v = _dot_t(xf, wv[...]) + bv[...]
    q3 = q.reshape(bb, S, C)
    k3 = k.reshape(bb, S, C)
    v3 = v.reshape(bb, S, C)
    ci = lax.broadcasted_iota(jnp.int32, (C, NHEAD), 0)
    hi = lax.broadcasted_iota(jnp.int32, (C, NHEAD), 1)
    hmat = (ci // HD == hi).astype(jnp.float32)
    scores = []
    for j in range(S):
        prod = (q3 * k3[:, j:j + 1, :]).reshape(bb * S, C)
        scores.append(jnp.dot(prod, hmat, preferred_element_type=jnp.float32))
    mx = scores[0]
    for j in range(1, S):
        mx = jnp.maximum(mx, scores[j])
    exps = [jnp.exp(s - mx) for s in scores]
    z = exps[0]
    for j in range(1, S):
        z = z + exps[j]
    o3 = jnp.zeros((bb, S, C), jnp.float32)
    for j in range(S):
        aj = exps[j] / z
        ajl = lax.dot_general(aj, hmat, (((1,), (1,)), ((), ())),
                              preferred_element_type=jnp.float32)
        o3 = o3 + ajl.reshape(bb, S, C) * v3[:, j:j + 1, :]
    of = o3.reshape(bb * S, C)
    of = _dot_t(of, wo[...]) + bo[...]
    h = _ln(xf + of, ln1g[...], ln1b[...])
    ff = _dot_t(jnp.maximum(_dot_t(h, w1[...]) + b1[...], 0.0), w2[...]) + b2[...]
    h = _ln(h + ff, ln2g[...], ln2b[...])
    out = _ln(h, tng[...], tnb[...])
    o_ref[...] = out.reshape(bb, S, C)


def _tab_transformer(x_tab, wq, wk, wv, wo, w1, w2, bq, bk, bv, bo, b1, b2,
                     ln1g, ln1b, ln2g, ln2b, tng, tnb):
    b = x_tab.shape[0]
    bb = 128
    grid = (b // bb,)
    full = lambda shp: pl.BlockSpec(shp, lambda i: (0,) * len(shp))
    row = lambda: pl.BlockSpec((1, C), lambda i: (0, 0))
    in_specs = [pl.BlockSpec((bb, S, C), lambda i: (i, 0, 0))] + \
        [full((C, C))] * 4 + [full((FFD, C)) for FFD in (C, C)] + \
        [row()] * 6 + [row()] * 6
    return pl.pallas_call(
        _tab_body,
        grid=grid,
        in_specs=in_specs,
        out_specs=pl.BlockSpec((bb, S, C), lambda i: (i, 0, 0)),
        out_shape=jax.ShapeDtypeStruct((b, S, C), jnp.float32),
    )(x_tab, wq, wk, wv, wo, w1, w2, bq, bk, bv, bo, b1, b2,
      ln1g, ln1b, ln2g, ln2b, tng, tnb)


def _pre_body(x_ref, pd, ps, preb, ad_ref, as_ref):
    x = x_ref[...]
    ad_ref[...] = _dot_t(x, pd[...]) + preb[...]
    as_ref[...] = _dot_t(x, ps[...])


def _node_pre(x_gnn, pd, ps, preb):
    n = x_gnn.shape[0]
    nb = 2000
    full = lambda shp: pl.BlockSpec(shp, lambda i: (0,) * len(shp))
    return pl.pallas_call(
        _pre_body,
        grid=(n // nb,),
        in_specs=[pl.BlockSpec((nb, C), lambda i: (i, 0)),
                  full((C, C)), full((C, C)), full((1, C))],
        out_specs=[pl.BlockSpec((nb, C), lambda i: (i, 0))] * 2,
        out_shape=[jax.ShapeDtypeStruct((n, C), jnp.float32)] * 2,
    )(x_gnn, pd, ps, preb)


def _edge1_body(ea_ref, gsum_ref, ew, eb, pe, msg_ref):
    e = _dot_t(ea_ref[...], ew[...]) + eb[...]
    msg_ref[...] = gsum_ref[...] + _dot_t(e, pe[...])


def _edge_msg(edge_attr, gsum, ew, eb, pe):
    e = edge_attr.shape[0]
    ebk = 2000
    full = lambda shp: pl.BlockSpec(shp, lambda i: (0,) * len(shp))
    return pl.pallas_call(
        _edge1_body,
        grid=(e // ebk,),
        in_specs=[pl.BlockSpec((ebk, C), lambda i: (i, 0)),
                  pl.BlockSpec((ebk, C), lambda i: (i, 0)),
                  full((C, C)), full((1, C)), full((C, C))],
        out_specs=pl.BlockSpec((ebk, C), lambda i: (i, 0)),
        out_shape=jax.ShapeDtypeStruct((e, C), jnp.float32),
    )(edge_attr, gsum, ew, eb, pe)


def _delta_body(cnt_ref, d_ref):
    c = cnt_ref[...]
    d_ref[...] = jnp.sum(jnp.log(c + 1.0), axis=0, keepdims=True) / c.shape[0]


def _delta(cnt2d):
    n = cnt2d.shape[0]
    return pl.pallas_call(
        _delta_body,
        in_specs=[pl.BlockSpec((n, 1), lambda: (0, 0))],
        out_specs=pl.BlockSpec((1, 1), lambda: (0, 0)),
        out_shape=jax.ShapeDtypeStruct((1, 1), jnp.float32),
    )(cnt2d)


def _node_body(x_ref, sum_ref, sq_ref, mxr_ref, mnr_ref, cnt_ref, delta_ref,
               p0, q1, q2, q3w, postb, linw, linb,
               out_ref, bns_ref, bnq_ref):
    i = pl.program_id(0)
    cnt = cnt_ref[...]
    cntc = jnp.maximum(cnt, 1.0)
    mean = sum_ref[...] / cntc
    mean2 = sq_ref[...] / cntc
    std = jnp.sqrt(jnp.maximum(mean2 - mean * mean, 0.0) + 1e-5)
    pos = cnt > 0.0
    mx = jnp.where(pos, mxr_ref[...], 0.0)
    mn = jnp.where(pos, mnr_ref[...], 0.0)
    agg = jnp.concatenate([mean, mx, mn, std], axis=-1)
    delta = delta_ref[0, 0]
    ldeg = jnp.log(cntc + 1.0)
    s1 = ldeg / delta
    s2 = delta / ldeg
    out = _dot_t(x_ref[...], p0[...]) + _dot_t(agg, q1[...]) \
        + _dot_t(agg, q2[...]) * s1 + _dot_t(agg, q3w[...]) * s2 + postb[...]
    out = _dot_t(out, linw[...]) + linb[...]
    out_ref[...] = out

    @pl.when(i == 0)
    def _init():
        bns_ref[...] = jnp.zeros_like(bns_ref)
        bnq_ref[...] = jnp.zeros_like(bnq_ref)

    bns_ref[...] += jnp.sum(out, axis=0, keepdims=True)
    bnq_ref[...] += jnp.sum(out * out, axis=0, keepdims=True)


def _node_mlp(x_gnn, sums, sumsq, mxr, mnr, cnt2d, delta,
              p0, q1, q2, q3w, postb, linw, linb):
    n = x_gnn.shape[0]
    nb = 2000
    full = lambda shp: pl.BlockSpec(shp, lambda i: (0,) * len(shp))
    blk = lambda w: pl.BlockSpec((nb, w), lambda i: (i, 0))
    return pl.pallas_call(
        _node_body,
        grid=(n // nb,),
        in_specs=[blk(C), blk(C), blk(C), blk(C), blk(C),
                  pl.BlockSpec((nb, 1), lambda i: (i, 0)), full((1, 1)),
                  full((C, C)), full((C, 4 * C)), full((C, 4 * C)),
                  full((C, 4 * C)), full((1, C)), full((C, C)), full((1, C))],
        out_specs=[blk(C), full((1, C)), full((1, C))],
        out_shape=[jax.ShapeDtypeStruct((n, C), jnp.float32),
                   jax.ShapeDtypeStruct((1, C), jnp.float32),
                   jax.ShapeDtypeStruct((1, C), jnp.float32)],
    )(x_gnn, sums, sumsq, mxr, mnr, cnt2d, delta,
      p0, q1, q2, q3w, postb, linw, linb)


def _bn_body(out_ref, bns_ref, bnq_ref, x_ref, bng, bnb, us_w, ud_w, eub1,
             xn_ref, us_ref, ud_ref):
    n_total = jnp.float32(10000.0)
    bm = bns_ref[...] / n_total
    bv = bnq_ref[...] / n_total - bm * bm
    o = (out_ref[...] - bm) / jnp.sqrt(bv + 1e-5) * bng[...] + bnb[...]
    xn = (x_ref[...] + jnp.maximum(o, 0.0)) * 0.5
    xn_ref[...] = xn
    us_ref[...] = _dot_t(xn, us_w[...]) + eub1[...]
    ud_ref[...] = _dot_t(xn, ud_w[...])


def _bn_apply(out2, bns, bnq, x_gnn, bng, bnb, us_w, ud_w, eub1):
    n = x_gnn.shape[0]
    nb = 2000
    full = lambda shp: pl.BlockSpec(shp, lambda i: (0,) * len(shp))
    blk = pl.BlockSpec((nb, C), lambda i: (i, 0))
    return pl.pallas_call(
        _bn_body,
        grid=(n // nb,),
        in_specs=[blk, full((1, C)), full((1, C)), blk,
                  full((1, C)), full((1, C)), full((C, C)), full((C, C)),
                  full((1, C))],
        out_specs=[blk, blk, blk],
        out_shape=[jax.ShapeDtypeStruct((n, C), jnp.float32)] * 3,
    )(out2, bns, bnq, x_gnn, bng, bnb, us_w, ud_w, eub1)


def _edge2_body(ea_ref, gu_ref, ue, euw2, eub2, o_ref):
    ea = ea_ref[...]
    h1 = jnp.maximum(gu_ref[...] + _dot_t(ea, ue[...]), 0.0)
    eh = _dot_t(h1, euw2[...]) + eub2[...]
    o_ref[...] = ea + 0.5 * eh


def _edge_update(edge_attr, gu, ue, euw2, eub2):
    e = edge_attr.shape[0]
    ebk = 2000
    full = lambda shp: pl.BlockSpec(shp, lambda i: (0,) * len(shp))
    return pl.pallas_call(
        _edge2_body,
        grid=(e // ebk,),
        in_specs=[pl.BlockSpec((ebk, C), lambda i: (i, 0)),
                  pl.BlockSpec((ebk, C), lambda i: (i, 0)),
                  full((C, C)), full((C, C)), full((1, C))],
        out_specs=pl.BlockSpec((ebk, C), lambda i: (i, 0)),
        out_shape=jax.ShapeDtypeStruct((e, C), jnp.float32),
    )(edge_attr, gu, ue, euw2, eub2)


_EDGES = 320000
_NNODES = 10000
_RNG = 160
_NPAD = 64 * _RNG
_SCCH = 1600
_NCHUNK = _EDGES // _SCCH
_NGRP = _SCCH // 16
_UNROLL = 4


def _seg_body(msg_hbm, dst_hbm, sums_hbm, sq_hbm, mx_hbm, mn_hbm, cnt_hbm,
              acc_s, acc_q, acc_mx, acc_mn, acc_c,
              dstbuf, idm, dstm, idx_a, idx_b, row_a, row_b, sem_a, sem_b):
    cidx = lax.axis_index("c")
    sidx = lax.axis_index("s")
    wid = sidx * 2 + cidx
    iota = lax.iota(jnp.int32, 16)
    zf = jnp.zeros((16,), jnp.float32)
    zi = jnp.zeros((16,), jnp.int32)
    onev = jnp.full((16,), 1, jnp.int32)
    oh0 = (iota == zi).astype(jnp.float32)
    neg = jnp.full((16,), -jnp.inf, jnp.float32)
    pos = jnp.full((16,), jnp.inf, jnp.float32)

    def _zidm(i, carry):
        idm[pl.ds(i * 16, 16)] = zi
        return carry

    lax.fori_loop(0, (_SCCH + 32) // 16, _zidm, 0)

    for h in range(2):
        rid = wid * 2 + h
        lo = rid * _RNG
        hi = lo + _RNG
        lov = jnp.full((16,), lo, jnp.int32)
        hiv = jnp.full((16,), hi, jnp.int32)

        def _zero(r, carry):
            for cg in range(8):
                sl = pl.ds(cg * 16, 16)
                acc_s[r, sl] = zf
                acc_q[r, sl] = zf
                acc_mx[r, sl] = neg
                acc_mn[r, sl] = pos
            return carry

        lax.fori_loop(0, _RNG, _zero, 0)
        for i in range((_RNG + 16) // 16):
            acc_c[pl.ds(i * 16, 16)] = zf

        def _chunk(ck, carry):
            base_e = ck * _SCCH
            pltpu.sync_copy(dst_hbm.at[pl.ds(base_e, _SCCH)], dstbuf)

            def _scan(g4, m):
                for u in range(_UNROLL):
                    g16 = (g4 * _UNROLL + u) * 16
                    dv = dstbuf[pl.ds(g16, 16)]
                    msk = (dv >= lov) & (dv < hiv)
                    ids = jnp.full((16,), base_e, jnp.int32) \
                        + jnp.full((16,), g16, jnp.int32) + iota
                    mi = msk.astype(jnp.int32)
                    csum = plsc.cumsum(mi)
                    trash_i = jnp.full((16,), _SCCH + 8, jnp.int32)
                    mv = jnp.full((16,), m, jnp.int32)
                    posn = jnp.where(msk, mv + csum - onev, trash_i)
                    plsc.store_scatter(idm, [posn], ids)
                    plsc.store_scatter(dstm, [posn], dv)
                    m = m + jnp.sum(mi)
                return m

            mtot = lax.fori_loop(0, _NGRP // _UNROLL, _scan, 0)
            idm[pl.ds(mtot, 16)] = zi
            dstm[pl.ds(mtot, 16)] = jnp.full((16,), -1, jnp.int32)
            dstm[pl.ds(mtot + 16, 16)] = jnp.full((16,), -1, jnp.int32)
            ngrp2 = (mtot + 15) // 16

            def _process(rowbuf, dvs):
                for l in range(16):
                    sel = iota == jnp.full((16,), l, jnp.int32)
                    dsc = jnp.sum(jnp.where(sel, dvs, zi))

                    @pl.when(dsc >= lo)
                    def _acc_one():
                        r = dsc - lo
                        for cg in range(8):
                            sl = pl.ds(cg * 16, 16)
                            rv = rowbuf[l, sl]
                            acc_s[r, sl] = acc_s[r, sl] + rv
                            acc_q[r, sl] = acc_q[r, sl] + rv * rv
                            acc_mx[r, sl] = jnp.maximum(acc_mx[r, sl], rv)
                            acc_mn[r, sl] = jnp.minimum(acc_mn[r, sl], rv)
                        acc_c[pl.ds(r, 16)] = acc_c[pl.ds(r, 16)] + oh0

            def _accum2(gp, carry2):
                ga = gp * 2
                idx_a[...] = idm[pl.ds(ga * 16, 16)]
                ha = pltpu.async_copy(msg_hbm.at[idx_a], row_a, sem_a)
                idx_b[...] = idm[pl.ds(ga * 16 + 16, 16)]
                hb = pltpu.async_copy(msg_hbm.at[idx_b], row_b, sem_b)
                ha.wait()
                _process(row_a, dstm[pl.ds(ga * 16, 16)])
                hb.wait()

                @pl.when(ga + 1 < ngrp2)
                def _phase_b():
                    _process(row_b, dstm[pl.ds(ga * 16 + 16, 16)])
                return carry2

            lax.fori_loop(0, (ngrp2 + 1) // 2, _accum2, 0)
            return carry

        lax.fori_loop(0, _NCHUNK, _chunk, 0)

        pltpu.sync_copy(acc_s, sums_hbm.at[pl.ds(lo, _RNG)])
        pltpu.sync_copy(acc_q, sq_hbm.at[pl.ds(lo, _RNG)])
        pltpu.sync_copy(acc_mx, mx_hbm.at[pl.ds(lo, _RNG)])
        pltpu.sync_copy(acc_mn, mn_hbm.at[pl.ds(lo, _RNG)])
        pltpu.sync_copy(acc_c.at[pl.ds(0, _RNG)], cnt_hbm.at[pl.ds(lo, _RNG)])


def _sc_segment_reduce(msg, dst):
    f = jnp.float32
    out = functools.partial(
        pl.kernel,
        mesh=plsc.VectorSubcoreMesh(core_axis_name="c", subcore_axis_name="s"),
        compiler_params=pltpu.CompilerParams(needs_layout_passes=False),
        out_type=[jax.ShapeDtypeStruct((_NPAD, C), f)] * 4
        + [jax.ShapeDtypeStruct((_NPAD,), f)],
        scratch_types=[
            pltpu.VMEM((_RNG, C), f),
            pltpu.VMEM((_RNG, C), f),
            pltpu.VMEM((_RNG, C), f),
            pltpu.VMEM((_RNG, C), f),
            pltpu.VMEM((_RNG + 16,), f),
            pltpu.VMEM((_SCCH,), jnp.int32),
            pltpu.VMEM((_SCCH + 32,), jnp.int32),
            pltpu.VMEM((_SCCH + 32,), jnp.int32),
            pltpu.VMEM((16,), jnp.int32),
            pltpu.VMEM((16,), jnp.int32),
            pltpu.VMEM((16, C), f),
            pltpu.VMEM((16, C), f),
            pltpu.SemaphoreType.DMA,
            pltpu.SemaphoreType.DMA,
        ],
    )(_seg_body)
    return out(msg, dst)


def kernel(x_tab, x_gnn, edge_index, edge_attr, target_edge_index,
           Wq, Wk, Wv, Wo, W1, W2, eW, preW, postW, linW, euW1, euW2,
           bq, bk, bv, bo, b1, b2, eb, preb, postb, linb, eub1, eub2,
           ln1_b, ln2_b, tn_b, bn_b, ln1_g, ln2_g, tn_g, bn_g):
    del target_edge_index
    n = x_gnn.shape[0]
    r = lambda v: v.reshape(1, -1)

    x_tab_out = _tab_transformer(
        x_tab, Wq, Wk, Wv, Wo, W1, W2, r(bq), r(bk), r(bv), r(bo), r(b1),
        r(b2), r(ln1_g), r(ln1_b), r(ln2_g), r(ln2_b), r(tn_g), r(tn_b))

    src, dst = edge_index[0], edge_index[1]

    pd, ps, pe = preW[:, :C], preW[:, C:2 * C], preW[:, 2 * C:]
    ad, as_ = _node_pre(x_gnn, pd, ps, r(preb))
    gsum = ad[dst] + as_[src]
    msg = _edge_msg(edge_attr, gsum, eW, r(eb), pe)

    sums_p, sumsq_p, mxr_p, mnr_p, cnt_p = _sc_segment_reduce(msg, dst)
    sums = sums_p[:n]
    sumsq = sumsq_p[:n]
    mxr = mxr_p[:n]
    mnr = mnr_p[:n]
    cnt2d = cnt_p[:n].reshape(n, 1)

    delta = _delta(cnt2d)

    p0 = postW[:, :C]
    q1 = postW[:, C:5 * C]
    q2 = postW[:, 5 * C:9 * C]
    q3w = postW[:, 9 * C:]
    out2, bns, bnq = _node_mlp(x_gnn, sums, sumsq, mxr, mnr, cnt2d, delta,
                               p0, q1, q2, q3w, r(postb), linW, r(linb))

    us_w, ud_w, ue = euW1[:, :C], euW1[:, C:2 * C], euW1[:, 2 * C:]
    x_gnn_new, us, ud = _bn_apply(out2, bns, bnq, x_gnn, r(bn_g), r(bn_b),
                                  us_w, ud_w, r(eub1))

    gu = us[src] + ud[dst]
    edge_attr_new = _edge_update(edge_attr, gu, ue, euW2, r(eub2))

    return (x_tab_out, x_gnn_new, edge_attr_new)


# speedup vs baseline: 1.4806x; 1.4806x over previous
import functools

import jax
import jax.numpy as jnp
from jax import lax
from jax.experimental import pallas as pl
from jax.experimental.pallas import tpu as pltpu
from jax.experimental.pallas import tpu_sc as plsc

C = 128
NHEAD = 8
HD = C // NHEAD
S = 14
NHID = 128


def _dot_t(x, w):
    return lax.dot_general(x, w, (((1,), (1,)), ((), ())),
                           preferred_element_type=jnp.float32)


def _ln(x, g, b, eps=1e-5):
    m = jnp.mean(x, -1, keepdims=True)
    v = jnp.mean((x - m) * (x - m), -1, keepdims=True)
    return (x - m) / jnp.sqrt(v + eps) * g + b


def _tab_body(x_ref, wq, wk, wv, wo, w1, w2, bq, bk, bv, bo, b1, b2,
              ln1g, ln1b, ln2g, ln2b, tng, tnb, o_ref):
    x = x_ref[...]
    bb = x.shape[0]
    xf = x.reshape(bb * S, C)
    q = (_dot_t(xf, wq[...]) + bq[...]) * (1.0 / jnp.sqrt(jnp.float32(HD)))
    k = _dot_t(xf, wk[...]) + bk[...]
    v = _dot_t(xf, wv[...]) + bv[...]
    q3 = q.reshape(bb, S, C)
    k3 = k.reshape(bb, S, C)
    v3 = v.reshape(bb, S, C)
    ci = lax.broadcasted_iota(jnp.int32, (C, NHEAD), 0)
    hi = lax.broadcasted_iota(jnp.int32, (C, NHEAD), 1)
    hmat = (ci // HD == hi).astype(jnp.float32)
    scores = []
    for j in range(S):
        prod = (q3 * k3[:, j:j + 1, :]).reshape(bb * S, C)
        scores.append(jnp.dot(prod, hmat, preferred_element_type=jnp.float32))
    mx = scores[0]
    for j in range(1, S):
        mx = jnp.maximum(mx, scores[j])
    exps = [jnp.exp(s - mx) for s in scores]
    z = exps[0]
    for j in range(1, S):
        z = z + exps[j]
    o3 = jnp.zeros((bb, S, C), jnp.float32)
    for j in range(S):
        aj = exps[j] / z
        ajl = lax.dot_general(aj, hmat, (((1,), (1,)), ((), ())),
                              preferred_element_type=jnp.float32)
        o3 = o3 + ajl.reshape(bb, S, C) * v3[:, j:j + 1, :]
    of = o3.reshape(bb * S, C)
    of = _dot_t(of, wo[...]) + bo[...]
    h = _ln(xf + of, ln1g[...], ln1b[...])
    ff = _dot_t(jnp.maximum(_dot_t(h, w1[...]) + b1[...], 0.0), w2[...]) + b2[...]
    h = _ln(h + ff, ln2g[...], ln2b[...])
    out = _ln(h, tng[...], tnb[...])
    o_ref[...] = out.reshape(bb, S, C)


def _tab_transformer(x_tab, wq, wk, wv, wo, w1, w2, bq, bk, bv, bo, b1, b2,
                     ln1g, ln1b, ln2g, ln2b, tng, tnb):
    b = x_tab.shape[0]
    bb = 128
    grid = (b // bb,)
    full = lambda shp: pl.BlockSpec(shp, lambda i: (0,) * len(shp))
    row = lambda: pl.BlockSpec((1, C), lambda i: (0, 0))
    in_specs = [pl.BlockSpec((bb, S, C), lambda i: (i, 0, 0))] + \
        [full((C, C))] * 4 + [full((FFD, C)) for FFD in (C, C)] + \
        [row()] * 6 + [row()] * 6
    return pl.pallas_call(
        _tab_body,
        grid=grid,
        in_specs=in_specs,
        out_specs=pl.BlockSpec((bb, S, C), lambda i: (i, 0, 0)),
        out_shape=jax.ShapeDtypeStruct((b, S, C), jnp.float32),
    )(x_tab, wq, wk, wv, wo, w1, w2, bq, bk, bv, bo, b1, b2,
      ln1g, ln1b, ln2g, ln2b, tng, tnb)


def _pre_body(x_ref, pd, ps, preb, ad_ref, as_ref):
    x = x_ref[...]
    ad_ref[...] = _dot_t(x, pd[...]) + preb[...]
    as_ref[...] = _dot_t(x, ps[...])


def _node_pre(x_gnn, pd, ps, preb):
    n = x_gnn.shape[0]
    nb = 2000
    full = lambda shp: pl.BlockSpec(shp, lambda i: (0,) * len(shp))
    return pl.pallas_call(
        _pre_body,
        grid=(n // nb,),
        in_specs=[pl.BlockSpec((nb, C), lambda i: (i, 0)),
                  full((C, C)), full((C, C)), full((1, C))],
        out_specs=[pl.BlockSpec((nb, C), lambda i: (i, 0))] * 2,
        out_shape=[jax.ShapeDtypeStruct((n, C), jnp.float32)] * 2,
    )(x_gnn, pd, ps, preb)


def _edge1_body(ea_ref, gsum_ref, ew, eb, pe, msg_ref):
    e = _dot_t(ea_ref[...], ew[...]) + eb[...]
    msg_ref[...] = gsum_ref[...] + _dot_t(e, pe[...])


def _edge_msg(edge_attr, gsum, ew, eb, pe):
    e = edge_attr.shape[0]
    ebk = 2000
    full = lambda shp: pl.BlockSpec(shp, lambda i: (0,) * len(shp))
    return pl.pallas_call(
        _edge1_body,
        grid=(e // ebk,),
        in_specs=[pl.BlockSpec((ebk, C), lambda i: (i, 0)),
                  pl.BlockSpec((ebk, C), lambda i: (i, 0)),
                  full((C, C)), full((1, C)), full((C, C))],
        out_specs=pl.BlockSpec((ebk, C), lambda i: (i, 0)),
        out_shape=jax.ShapeDtypeStruct((e, C), jnp.float32),
    )(edge_attr, gsum, ew, eb, pe)


def _delta_body(cnt_ref, d_ref):
    c = cnt_ref[...]
    d_ref[...] = jnp.sum(jnp.log(c + 1.0), axis=0, keepdims=True) / c.shape[0]


def _delta(cnt2d):
    n = cnt2d.shape[0]
    return pl.pallas_call(
        _delta_body,
        in_specs=[pl.BlockSpec((n, 1), lambda: (0, 0))],
        out_specs=pl.BlockSpec((1, 1), lambda: (0, 0)),
        out_shape=jax.ShapeDtypeStruct((1, 1), jnp.float32),
    )(cnt2d)


def _node_body(x_ref, sum_ref, sq_ref, mxr_ref, mnr_ref, cnt_ref, delta_ref,
               p0, q1, q2, q3w, postb, linw, linb,
               out_ref, bns_ref, bnq_ref):
    i = pl.program_id(0)
    cnt = cnt_ref[...]
    cntc = jnp.maximum(cnt, 1.0)
    mean = sum_ref[...] / cntc
    mean2 = sq_ref[...] / cntc
    std = jnp.sqrt(jnp.maximum(mean2 - mean * mean, 0.0) + 1e-5)
    pos = cnt > 0.0
    mx = jnp.where(pos, mxr_ref[...], 0.0)
    mn = jnp.where(pos, mnr_ref[...], 0.0)
    agg = jnp.concatenate([mean, mx, mn, std], axis=-1)
    delta = delta_ref[0, 0]
    ldeg = jnp.log(cntc + 1.0)
    s1 = ldeg / delta
    s2 = delta / ldeg
    out = _dot_t(x_ref[...], p0[...]) + _dot_t(agg, q1[...]) \
        + _dot_t(agg, q2[...]) * s1 + _dot_t(agg, q3w[...]) * s2 + postb[...]
    out = _dot_t(out, linw[...]) + linb[...]
    out_ref[...] = out

    @pl.when(i == 0)
    def _init():
        bns_ref[...] = jnp.zeros_like(bns_ref)
        bnq_ref[...] = jnp.zeros_like(bnq_ref)

    bns_ref[...] += jnp.sum(out, axis=0, keepdims=True)
    bnq_ref[...] += jnp.sum(out * out, axis=0, keepdims=True)


def _node_mlp(x_gnn, sums, sumsq, mxr, mnr, cnt2d, delta,
              p0, q1, q2, q3w, postb, linw, linb):
    n = x_gnn.shape[0]
    nb = 2000
    full = lambda shp: pl.BlockSpec(shp, lambda i: (0,) * len(shp))
    blk = lambda w: pl.BlockSpec((nb, w), lambda i: (i, 0))
    return pl.pallas_call(
        _node_body,
        grid=(n // nb,),
        in_specs=[blk(C), blk(C), blk(C), blk(C), blk(C),
                  pl.BlockSpec((nb, 1), lambda i: (i, 0)), full((1, 1)),
                  full((C, C)), full((C, 4 * C)), full((C, 4 * C)),
                  full((C, 4 * C)), full((1, C)), full((C, C)), full((1, C))],
        out_specs=[blk(C), full((1, C)), full((1, C))],
        out_shape=[jax.ShapeDtypeStruct((n, C), jnp.float32),
                   jax.ShapeDtypeStruct((1, C), jnp.float32),
                   jax.ShapeDtypeStruct((1, C), jnp.float32)],
    )(x_gnn, sums, sumsq, mxr, mnr, cnt2d, delta,
      p0, q1, q2, q3w, postb, linw, linb)


def _bn_body(out_ref, bns_ref, bnq_ref, x_ref, bng, bnb, us_w, ud_w, eub1,
             xn_ref, us_ref, ud_ref):
    n_total = jnp.float32(10000.0)
    bm = bns_ref[...] / n_total
    bv = bnq_ref[...] / n_total - bm * bm
    o = (out_ref[...] - bm) / jnp.sqrt(bv + 1e-5) * bng[...] + bnb[...]
    xn = (x_ref[...] + jnp.maximum(o, 0.0)) * 0.5
    xn_ref[...] = xn
    us_ref[...] = _dot_t(xn, us_w[...]) + eub1[...]
    ud_ref[...] = _dot_t(xn, ud_w[...])


def _bn_apply(out2, bns, bnq, x_gnn, bng, bnb, us_w, ud_w, eub1):
    n = x_gnn.shape[0]
    nb = 2000
    full = lambda shp: pl.BlockSpec(shp, lambda i: (0,) * len(shp))
    blk = pl.BlockSpec((nb, C), lambda i: (i, 0))
    return pl.pallas_call(
        _bn_body,
        grid=(n // nb,),
        in_specs=[blk, full((1, C)), full((1, C)), blk,
                  full((1, C)), full((1, C)), full((C, C)), full((C, C)),
                  full((1, C))],
        out_specs=[blk, blk, blk],
        out_shape=[jax.ShapeDtypeStruct((n, C), jnp.float32)] * 3,
    )(out2, bns, bnq, x_gnn, bng, bnb, us_w, ud_w, eub1)


def _edge2_body(ea_ref, gu_ref, ue, euw2, eub2, o_ref):
    ea = ea_ref[...]
    h1 = jnp.maximum(gu_ref[...] + _dot_t(ea, ue[...]), 0.0)
    eh = _dot_t(h1, euw2[...]) + eub2[...]
    o_ref[...] = ea + 0.5 * eh


def _edge_update(edge_attr, gu, ue, euw2, eub2):
    e = edge_attr.shape[0]
    ebk = 2000
    full = lambda shp: pl.BlockSpec(shp, lambda i: (0,) * len(shp))
    return pl.pallas_call(
        _edge2_body,
        grid=(e // ebk,),
        in_specs=[pl.BlockSpec((ebk, C), lambda i: (i, 0)),
                  pl.BlockSpec((ebk, C), lambda i: (i, 0)),
                  full((C, C)), full((C, C)), full((1, C))],
        out_specs=pl.BlockSpec((ebk, C), lambda i: (i, 0)),
        out_shape=jax.ShapeDtypeStruct((e, C), jnp.float32),
    )(edge_attr, gu, ue, euw2, eub2)


_EDGES = 320000
_NNODES = 10000
_RNG = 160
_NPAD = 64 * _RNG
_SCCH = 8000
_NCHUNK = _EDGES // _SCCH
_NGRP = _SCCH // 16
_UNROLL = 4
_GB = 64


def _seg_body(msg_hbm, dst_hbm, sums_hbm, sq_hbm, mx_hbm, mn_hbm, cnt_hbm,
              acc_s, acc_q, acc_mx, acc_mn, acc_c,
              dstbuf, idm, dstm, rowbuf, sem):
    cidx = lax.axis_index("c")
    sidx = lax.axis_index("s")
    wid = sidx * 2 + cidx
    iota = lax.iota(jnp.int32, 16)
    zf = jnp.zeros((16,), jnp.float32)
    zi = jnp.zeros((16,), jnp.int32)
    onev = jnp.full((16,), 1, jnp.int32)
    oh0 = (iota == zi).astype(jnp.float32)
    neg = jnp.full((16,), -jnp.inf, jnp.float32)
    pos = jnp.full((16,), jnp.inf, jnp.float32)
    negone = jnp.full((16,), -1, jnp.int32)

    def _zidm(i, carry):
        idm[pl.ds(i * 16, 16)] = zi
        return carry

    lax.fori_loop(0, (_SCCH + _GB) // 16, _zidm, 0)

    for h in range(2):
        rid = wid * 2 + h
        lo = rid * _RNG
        hi = lo + _RNG
        lov = jnp.full((16,), lo, jnp.int32)
        hiv = jnp.full((16,), hi, jnp.int32)

        def _zero(r, carry):
            for cg in range(8):
                sl = pl.ds(cg * 16, 16)
                acc_s[r, sl] = zf
                acc_q[r, sl] = zf
                acc_mx[r, sl] = neg
                acc_mn[r, sl] = pos
            return carry

        lax.fori_loop(0, _RNG, _zero, 0)
        for i in range((_RNG + 16) // 16):
            acc_c[pl.ds(i * 16, 16)] = zf

        def _chunk(ck, carry):
            base_e = ck * _SCCH
            pltpu.sync_copy(dst_hbm.at[pl.ds(base_e, _SCCH)], dstbuf)

            def _scan(g4, m):
                for u in range(_UNROLL):
                    g16 = 16 * _UNROLL * g4 + 16 * u
                    dv = dstbuf[pl.ds(g16, 16)]
                    msk = (dv >= lov) & (dv < hiv)
                    ids = jnp.full((16,), base_e, jnp.int32) \
                        + jnp.full((16,), g16, jnp.int32) + iota
                    mi = msk.astype(jnp.int32)
                    csum = plsc.cumsum(mi)
                    trash_i = jnp.full((16,), _SCCH + _GB - 8, jnp.int32)
                    mv = jnp.full((16,), m, jnp.int32)
                    posn = jnp.where(msk, mv + csum - onev, trash_i)
                    plsc.store_scatter(idm, [posn], ids)
                    plsc.store_scatter(dstm, [posn], dv)
                    m = m + jnp.sum(mi)
                return m

            mtot = lax.fori_loop(0, _NGRP // _UNROLL, _scan, 0)
            for p in range(_GB // 16):
                dstm[pl.ds(mtot + 16 * p, 16)] = negone
            ngrp2 = (mtot + 15) // 16
            ngrp64 = (mtot + _GB - 1) // _GB

            def _accum(g64, carry2):
                pltpu.async_copy(
                    msg_hbm.at[idm.at[pl.ds(g64 * _GB, _GB)]], rowbuf,
                    sem).wait()

                def _sub(sub, carry3):
                    gq = g64 * 4 + sub
                    dvs = dstm[pl.ds(gq * 16, 16)]

                    @pl.when(gq < ngrp2)
                    def _grp():
                        for l in range(16):
                            sel = iota == jnp.full((16,), l, jnp.int32)
                            dsc = jnp.sum(jnp.where(sel, dvs, zi))

                            @pl.when(dsc >= lo)
                            def _acc_one():
                                r = dsc - lo
                                rl = sub * 16 + l
                                for cg in range(8):
                                    sl = pl.ds(cg * 16, 16)
                                    rv = rowbuf[rl, sl]
                                    acc_s[r, sl] = acc_s[r, sl] + rv
                                    acc_q[r, sl] = acc_q[r, sl] + rv * rv
                                    acc_mx[r, sl] = jnp.maximum(
                                        acc_mx[r, sl], rv)
                                    acc_mn[r, sl] = jnp.minimum(
                                        acc_mn[r, sl], rv)
                                acc_c[pl.ds(r, 16)] = \
                                    acc_c[pl.ds(r, 16)] + oh0
                    return carry3

                lax.fori_loop(0, 4, _sub, 0)
                return carry2

            lax.fori_loop(0, ngrp64, _accum, 0)
            return carry

        lax.fori_loop(0, _NCHUNK, _chunk, 0)

        pltpu.sync_copy(acc_s, sums_hbm.at[pl.ds(lo, _RNG)])
        pltpu.sync_copy(acc_q, sq_hbm.at[pl.ds(lo, _RNG)])
        pltpu.sync_copy(acc_mx, mx_hbm.at[pl.ds(lo, _RNG)])
        pltpu.sync_copy(acc_mn, mn_hbm.at[pl.ds(lo, _RNG)])
        pltpu.sync_copy(acc_c.at[pl.ds(0, _RNG)], cnt_hbm.at[pl.ds(lo, _RNG)])


def _sc_segment_reduce(msg, dst):
    f = jnp.float32
    out = functools.partial(
        pl.kernel,
        mesh=plsc.VectorSubcoreMesh(core_axis_name="c", subcore_axis_name="s"),
        compiler_params=pltpu.CompilerParams(needs_layout_passes=False),
        out_type=[jax.ShapeDtypeStruct((_NPAD, C), f)] * 4
        + [jax.ShapeDtypeStruct((_NPAD,), f)],
        scratch_types=[
            pltpu.VMEM((_RNG, C), f),
            pltpu.VMEM((_RNG, C), f),
            pltpu.VMEM((_RNG, C), f),
            pltpu.VMEM((_RNG, C), f),
            pltpu.VMEM((_RNG + 16,), f),
            pltpu.VMEM((_SCCH,), jnp.int32),
            pltpu.VMEM((_SCCH + _GB,), jnp.int32),
            pltpu.VMEM((_SCCH + _GB,), jnp.int32),
            pltpu.VMEM((_GB, C), f),
            pltpu.SemaphoreType.DMA,
        ],
    )(_seg_body)
    return out(msg, dst)


def kernel(x_tab, x_gnn, edge_index, edge_attr, target_edge_index,
           Wq, Wk, Wv, Wo, W1, W2, eW, preW, postW, linW, euW1, euW2,
           bq, bk, bv, bo, b1, b2, eb, preb, postb, linb, eub1, eub2,
           ln1_b, ln2_b, tn_b, bn_b, ln1_g, ln2_g, tn_g, bn_g):
    del target_edge_index
    n = x_gnn.shape[0]
    r = lambda v: v.reshape(1, -1)

    x_tab_out = _tab_transformer(
        x_tab, Wq, Wk, Wv, Wo, W1, W2, r(bq), r(bk), r(bv), r(bo), r(b1),
        r(b2), r(ln1_g), r(ln1_b), r(ln2_g), r(ln2_b), r(tn_g), r(tn_b))

    src, dst = edge_index[0], edge_index[1]

    pd, ps, pe = preW[:, :C], preW[:, C:2 * C], preW[:, 2 * C:]
    ad, as_ = _node_pre(x_gnn, pd, ps, r(preb))
    gsum = ad[dst] + as_[src]
    msg = _edge_msg(edge_attr, gsum, eW, r(eb), pe)

    sums_p, sumsq_p, mxr_p, mnr_p, cnt_p = _sc_segment_reduce(msg, dst)
    sums = sums_p[:n]
    sumsq = sumsq_p[:n]
    mxr = mxr_p[:n]
    mnr = mnr_p[:n]
    cnt2d = cnt_p[:n].reshape(n, 1)

    delta = _delta(cnt2d)

    p0 = postW[:, :C]
    q1 = postW[:, C:5 * C]
    q2 = postW[:, 5 * C:9 * C]
    q3w = postW[:, 9 * C:]
    out2, bns, bnq = _node_mlp(x_gnn, sums, sumsq, mxr, mnr, cnt2d, delta,
                               p0, q1, q2, q3w, r(postb), linW, r(linb))

    us_w, ud_w, ue = euW1[:, :C], euW1[:, C:2 * C], euW1[:, 2 * C:]
    x_gnn_new, us, ud = _bn_apply(out2, bns, bnq, x_gnn, r(bn_g), r(bn_b),
                                  us_w, ud_w, r(eub1))

    gu = us[src] + ud[dst]
    edge_attr_new = _edge_update(edge_attr, gu, ue, euW2, r(eub2))

    return (x_tab_out, x_gnn_new, edge_attr_new)
